# baseline (speedup 1.0000x reference)
import functools

import jax
import jax.numpy as jnp
from jax import lax
from jax.experimental import pallas as pl
from jax.experimental.pallas import tpu as pltpu
from jax.experimental.pallas import tpu_sc as plsc

N = 10000
E = 320000
DX = 128
H = 128
DG = 32
BN_EPS = 1e-3

NC = 2
NS = 16
NW = NC * NS
EPW = E // NW
CHUNK = 128
NCHT = E // CHUNK
NCHUNK = NCHT // NW
NCREM = NCHT % NW
IBATCH = 16
RPT = 624
TAIL = N - NS * RPT
ZB = 16


def _sc_scatter_body(ei_hbm, ef_hbm, out_hbm,
                     idx_v, buf0, buf1, zbuf, msg_sh, sem0, sem1, semz, semi):
    c = lax.axis_index("c")
    s = lax.axis_index("s")
    wid = c * NS + s

    def chunk_src(j):
        return ef_hbm.at[pl.ds((wid + NW * j) * CHUNK, CHUNK)]

    def idx_src(j):
        return ei_hbm.at[pl.ds(0, 1), pl.ds((wid + NW * j) * CHUNK, CHUNK)]

    pltpu.async_copy(chunk_src(0), buf0, sem0)
    pltpu.async_copy(chunk_src(1), buf1, sem1)

    for r in range(ZB):
        for l in range(H // 16):
            zbuf[r, pl.ds(l * 16, 16)] = jnp.zeros((16,), jnp.float32)

    def zfire(k, carry):
        pltpu.async_copy(zbuf, msg_sh.at[pl.ds(s * RPT + k * ZB, ZB)], semz)
        return carry

    lax.fori_loop(0, RPT // ZB, zfire, 0)

    @pl.when(s == NS - 1)
    def _():
        pltpu.async_copy(zbuf, msg_sh.at[pl.ds(NS * RPT, TAIL)], semz)

    def stage(i, carry):
        def fire(k, carry2):
            pltpu.async_copy(idx_src(i * IBATCH + k),
                             idx_v.at[pl.ds(i * IBATCH + k, 1)], semi)
            return carry2

        lax.fori_loop(0, IBATCH, fire, 0)

        def drain(k, carry2):
            pltpu.make_async_copy(idx_src(0), idx_v.at[pl.ds(0, 1)],
                                  semi).wait()
            return carry2

        lax.fori_loop(0, IBATCH, drain, 0)
        return carry

    lax.fori_loop(0, NCHUNK // IBATCH, stage, 0)
    for j in range(NCHUNK - NCHUNK % IBATCH, NCHUNK):
        pltpu.async_copy(idx_src(j), idx_v.at[pl.ds(j, 1)], semi)

    @pl.when(wid < NCREM)
    def _():
        pltpu.async_copy(idx_src(NCHUNK), idx_v.at[pl.ds(NCHUNK, 1)], semi)

    for j in range(NCHUNK - NCHUNK % IBATCH, NCHUNK):
        pltpu.make_async_copy(idx_src(0), idx_v.at[pl.ds(0, 1)], semi).wait()

    @pl.when(wid < NCREM)
    def _():
        pltpu.make_async_copy(idx_src(0), idx_v.at[pl.ds(0, 1)], semi).wait()

    def zdrain(k, carry):
        pltpu.make_async_copy(zbuf, msg_sh.at[pl.ds(0, ZB)], semz).wait()
        return carry

    lax.fori_loop(0, RPT // ZB, zdrain, 0)

    @pl.when(s == NS - 1)
    def _():
        pltpu.make_async_copy(zbuf, msg_sh.at[pl.ds(0, ZB)], semz).wait()

    plsc.subcore_barrier()

    def body(i, carry):
        for b, (buf, sem) in enumerate(((buf0, sem0), (buf1, sem1))):
            j = 2 * i + b
            pltpu.make_async_copy(chunk_src(j), buf, sem).wait()
            pltpu.sync_copy(buf, msg_sh.at[idx_v.at[j]], add=True)

            @pl.when(wid + NW * (j + 2) < NCHT)
            def _():
                pltpu.async_copy(chunk_src(j + 2), buf, sem)
        return carry

    lax.fori_loop(0, NCHUNK // 2, body, 0)
    @pl.when(wid < NCREM)
    def _():
        pltpu.make_async_copy(chunk_src(NCHUNK), buf0, sem0).wait()
        pltpu.sync_copy(buf0, msg_sh.at[idx_v.at[NCHUNK]], add=True)

    plsc.subcore_barrier()
    pltpu.sync_copy(msg_sh.at[pl.ds(s * RPT, RPT)],
                    out_hbm.at[pl.ds(c * N + s * RPT, RPT)])

    @pl.when(s == NS - 1)
    def _():
        pltpu.sync_copy(msg_sh.at[pl.ds(NS * RPT, TAIL)],
                        out_hbm.at[pl.ds(c * N + NS * RPT, TAIL)])


_sc_scatter = functools.partial(
    pl.kernel,
    out_type=jax.ShapeDtypeStruct((NC * N, H), jnp.float32),
    mesh=plsc.VectorSubcoreMesh(core_axis_name="c", subcore_axis_name="s"),
    scratch_types=[
        pltpu.VMEM((NCHUNK + 1, CHUNK), jnp.int32),
        pltpu.VMEM((CHUNK, H), jnp.float32),
        pltpu.VMEM((CHUNK, H), jnp.float32),
        pltpu.VMEM((ZB, H), jnp.float32),
        pltpu.VMEM_SHARED((N, H), jnp.float32),
        pltpu.SemaphoreType.DMA,
        pltpu.SemaphoreType.DMA,
        pltpu.SemaphoreType.DMA,
        pltpu.SemaphoreType.DMA,
    ],
)(_sc_scatter_body)


def _softplus(v):
    return jnp.maximum(v, 0.0) + jnp.log1p(jnp.exp(-jnp.abs(v)))


def _dense_body(x_ref, p_ref, gf_ref, wc_ref, bc_ref,
                w1a_ref, b1a_ref, w2a_ref, b2a_ref,
                ga_a_ref, be_a_ref, mu_a_ref, va_a_ref,
                w1b_ref, b1b_ref, w2b_ref, b2b_ref,
                ga_b_ref, be_b_ref, mu_b_ref, va_b_ref,
                o_ref):
    f32 = jnp.float32
    wc = wc_ref[...]
    acc = jnp.dot(x_ref[...], wc[0:DX], preferred_element_type=f32)
    acc += jnp.dot(p_ref[0], wc[DX:DX + H], preferred_element_type=f32)
    acc += jnp.dot(p_ref[1], wc[DX:DX + H], preferred_element_type=f32)
    acc += jnp.dot(gf_ref[...], wc[DX + H:], preferred_element_type=f32)
    acc += bc_ref[...]
    out = _softplus(acc)
    initial = out
    blocks = (
        (w1a_ref, b1a_ref, w2a_ref, b2a_ref, ga_a_ref, be_a_ref, mu_a_ref,
         va_a_ref),
        (w1b_ref, b1b_ref, w2b_ref, b2b_ref, ga_b_ref, be_b_ref, mu_b_ref,
         va_b_ref),
    )
    for (w1, b1, w2, b2, ga, be, mu, va) in blocks:
        scale = ga[...] * lax.rsqrt(va[...] + BN_EPS)
        w2s = w2[...] * scale
        b2s = (b2[...] - mu[...]) * scale + be[...]
        h = _softplus(jnp.dot(out, w1[...], preferred_element_type=f32)
                      + b1[...])
        out = out + jnp.dot(h, w2s, preferred_element_type=f32) + b2s
    o_ref[...] = out + initial


BLK = 2000


def _dense(x, p, gf, wc, bc, *rest):
    full = lambda shape: pl.BlockSpec(shape, lambda i: (0,) * len(shape))
    in_specs = [
        pl.BlockSpec((BLK, DX), lambda i: (i, 0)),
        pl.BlockSpec((NC, BLK, H), lambda i: (0, i, 0)),
        full((1, DG)),
        full((DX + H + DG, H)),
        full((1, H)),
    ]
    for _ in range(2):
        in_specs += [full((H, H)), full((1, H)), full((H, H)), full((1, H)),
                     full((1, H)), full((1, H)), full((1, H)), full((1, H))]
    return pl.pallas_call(
        _dense_body,
        grid=(N // BLK,),
        in_specs=in_specs,
        out_specs=pl.BlockSpec((BLK, H), lambda i: (i, 0)),
        out_shape=jax.ShapeDtypeStruct((N, H), jnp.float32),
    )(x, p, gf, wc, bc, *rest)


def kernel(x, edge_index, edge_features, global_feat, W_c, b_c,
           W1_0, b1_0, W2_0, b2_0, gamma_0, beta_0, mean_0, var_0,
           W1_1, b1_1, W2_1, b2_1, gamma_1, beta_1, mean_1, var_1):
    p = _sc_scatter(edge_index, edge_features).reshape(NC, N, H)
    r = lambda a: a.reshape(1, H)
    return _dense(x, p, global_feat, W_c, r(b_c),
                  W1_0, r(b1_0), W2_0, r(b2_0),
                  r(gamma_0), r(beta_0), r(mean_0), r(var_0),
                  W1_1, r(b1_1), W2_1, r(b2_1),
                  r(gamma_1), r(beta_1), r(mean_1), r(var_1))


# speedup vs baseline: 1.0165x; 1.0165x over previous
import functools

import jax
import jax.numpy as jnp
from jax import lax
from jax.experimental import pallas as pl
from jax.experimental.pallas import tpu as pltpu
from jax.experimental.pallas import tpu_sc as plsc

N = 10000
E = 320000
DX = 128
H = 128
DG = 32
BN_EPS = 1e-3

NC = 2
NS = 16
NW = NC * NS
EPW = E // NW
CHUNK = 128
NCHT = E // CHUNK
NCHUNK = NCHT // NW
NCREM = NCHT % NW
IBATCH = 16
RPT = 624
TAIL = N - NS * RPT
ZB = 16


def _sc_scatter_body(ei_hbm, ef_hbm, out_hbm,
                     idx_v, buf0, buf1, zbuf, msg_sh, sem0, sem1, semz, semi):
    c = lax.axis_index("c")
    s = lax.axis_index("s")
    wid = c * NS + s

    def chunk_src(j):
        return ef_hbm.at[pl.ds((wid + NW * j) * CHUNK, CHUNK)]

    def idx_src(j):
        return ei_hbm.at[pl.ds(0, 1), pl.ds((wid + NW * j) * CHUNK, CHUNK)]

    pltpu.async_copy(chunk_src(0), buf0, sem0)
    pltpu.async_copy(chunk_src(1), buf1, sem1)

    for r in range(ZB):
        for l in range(H // 16):
            zbuf[r, pl.ds(l * 16, 16)] = jnp.zeros((16,), jnp.float32)

    def zfire(k, carry):
        pltpu.async_copy(zbuf, msg_sh.at[pl.ds(s * RPT + k * ZB, ZB)], semz)
        return carry

    lax.fori_loop(0, RPT // ZB, zfire, 0)

    @pl.when(s == NS - 1)
    def _():
        pltpu.async_copy(zbuf, msg_sh.at[pl.ds(NS * RPT, TAIL)], semz)

    def stage(i, carry):
        def fire(k, carry2):
            pltpu.async_copy(idx_src(i * IBATCH + k),
                             idx_v.at[pl.ds(i * IBATCH + k, 1)], semi)
            return carry2

        lax.fori_loop(0, IBATCH, fire, 0)

        def drain(k, carry2):
            pltpu.make_async_copy(idx_src(0), idx_v.at[pl.ds(0, 1)],
                                  semi).wait()
            return carry2

        lax.fori_loop(0, IBATCH, drain, 0)
        return carry

    lax.fori_loop(0, NCHUNK // IBATCH, stage, 0)
    for j in range(NCHUNK - NCHUNK % IBATCH, NCHUNK):
        pltpu.async_copy(idx_src(j), idx_v.at[pl.ds(j, 1)], semi)

    @pl.when(wid < NCREM)
    def _():
        pltpu.async_copy(idx_src(NCHUNK), idx_v.at[pl.ds(NCHUNK, 1)], semi)

    for j in range(NCHUNK - NCHUNK % IBATCH, NCHUNK):
        pltpu.make_async_copy(idx_src(0), idx_v.at[pl.ds(0, 1)], semi).wait()

    @pl.when(wid < NCREM)
    def _():
        pltpu.make_async_copy(idx_src(0), idx_v.at[pl.ds(0, 1)], semi).wait()

    def zdrain(k, carry):
        pltpu.make_async_copy(zbuf, msg_sh.at[pl.ds(0, ZB)], semz).wait()
        return carry

    lax.fori_loop(0, RPT // ZB, zdrain, 0)

    @pl.when(s == NS - 1)
    def _():
        pltpu.make_async_copy(zbuf, msg_sh.at[pl.ds(0, ZB)], semz).wait()

    plsc.subcore_barrier()

    def body(i, carry):
        for b, (buf, sem) in enumerate(((buf0, sem0), (buf1, sem1))):
            j = 2 * i + b
            pltpu.make_async_copy(chunk_src(j), buf, sem).wait()
            pltpu.sync_copy(buf, msg_sh.at[idx_v.at[j]], add=True)

            @pl.when(wid + NW * (j + 2) < NCHT)
            def _():
                pltpu.async_copy(chunk_src(j + 2), buf, sem)
        return carry

    lax.fori_loop(0, NCHUNK // 2, body, 0)
    @pl.when(wid < NCREM)
    def _():
        pltpu.make_async_copy(chunk_src(NCHUNK), buf0, sem0).wait()
        pltpu.sync_copy(buf0, msg_sh.at[idx_v.at[NCHUNK]], add=True)

    plsc.subcore_barrier()
    pltpu.sync_copy(msg_sh.at[pl.ds(s * RPT, RPT)],
                    out_hbm.at[pl.ds(c * N + s * RPT, RPT)])

    @pl.when(s == NS - 1)
    def _():
        pltpu.sync_copy(msg_sh.at[pl.ds(NS * RPT, TAIL)],
                        out_hbm.at[pl.ds(c * N + NS * RPT, TAIL)])


_sc_scatter = functools.partial(
    pl.kernel,
    out_type=jax.ShapeDtypeStruct((NC * N, H), jnp.float32),
    mesh=plsc.VectorSubcoreMesh(core_axis_name="c", subcore_axis_name="s"),
    scratch_types=[
        pltpu.VMEM((NCHUNK + 1, CHUNK), jnp.int32),
        pltpu.VMEM((CHUNK, H), jnp.float32),
        pltpu.VMEM((CHUNK, H), jnp.float32),
        pltpu.VMEM((ZB, H), jnp.float32),
        pltpu.VMEM_SHARED((N, H), jnp.float32),
        pltpu.SemaphoreType.DMA,
        pltpu.SemaphoreType.DMA,
        pltpu.SemaphoreType.DMA,
        pltpu.SemaphoreType.DMA,
    ],
)(_sc_scatter_body)


def _softplus(v):
    return jnp.maximum(v, 0.0) + jnp.log1p(jnp.exp(-jnp.abs(v)))


def _dense_body(x_ref, p_ref, gf_ref, wc_ref, bc_ref,
                w1a_ref, b1a_ref, w2a_ref, b2a_ref,
                ga_a_ref, be_a_ref, mu_a_ref, va_a_ref,
                w1b_ref, b1b_ref, w2b_ref, b2b_ref,
                ga_b_ref, be_b_ref, mu_b_ref, va_b_ref,
                o_ref):
    f32 = jnp.float32
    wc = wc_ref[...]
    acc = jnp.dot(x_ref[...], wc[0:DX], preferred_element_type=f32)
    acc += jnp.dot(p_ref[0] + p_ref[1], wc[DX:DX + H],
                   preferred_element_type=f32)
    acc += jnp.dot(gf_ref[...], wc[DX + H:], preferred_element_type=f32)
    acc += bc_ref[...]
    out = _softplus(acc)
    initial = out
    blocks = (
        (w1a_ref, b1a_ref, w2a_ref, b2a_ref, ga_a_ref, be_a_ref, mu_a_ref,
         va_a_ref),
        (w1b_ref, b1b_ref, w2b_ref, b2b_ref, ga_b_ref, be_b_ref, mu_b_ref,
         va_b_ref),
    )
    for (w1, b1, w2, b2, ga, be, mu, va) in blocks:
        scale = ga[...] * lax.rsqrt(va[...] + BN_EPS)
        w2s = w2[...] * scale
        b2s = (b2[...] - mu[...]) * scale + be[...]
        h = _softplus(jnp.dot(out, w1[...], preferred_element_type=f32)
                      + b1[...])
        out = out + jnp.dot(h, w2s, preferred_element_type=f32) + b2s
    o_ref[...] = out + initial


BLK = 2000


def _dense(x, p, gf, wc, bc, *rest):
    full = lambda shape: pl.BlockSpec(shape, lambda i: (0,) * len(shape))
    in_specs = [
        pl.BlockSpec((BLK, DX), lambda i: (i, 0)),
        pl.BlockSpec((NC, BLK, H), lambda i: (0, i, 0)),
        full((1, DG)),
        full((DX + H + DG, H)),
        full((1, H)),
    ]
    for _ in range(2):
        in_specs += [full((H, H)), full((1, H)), full((H, H)), full((1, H)),
                     full((1, H)), full((1, H)), full((1, H)), full((1, H))]
    return pl.pallas_call(
        _dense_body,
        grid=(N // BLK,),
        in_specs=in_specs,
        out_specs=pl.BlockSpec((BLK, H), lambda i: (i, 0)),
        out_shape=jax.ShapeDtypeStruct((N, H), jnp.float32),
    )(x, p, gf, wc, bc, *rest)


def kernel(x, edge_index, edge_features, global_feat, W_c, b_c,
           W1_0, b1_0, W2_0, b2_0, gamma_0, beta_0, mean_0, var_0,
           W1_1, b1_1, W2_1, b2_1, gamma_1, beta_1, mean_1, var_1):
    p = _sc_scatter(edge_index, edge_features).reshape(NC, N, H)
    r = lambda a: a.reshape(1, H)
    return _dense(x, p, global_feat, W_c, r(b_c),
                  W1_0, r(b1_0), W2_0, r(b2_0),
                  r(gamma_0), r(beta_0), r(mean_0), r(var_0),
                  W1_1, r(b1_1), W2_1, r(b2_1),
                  r(gamma_1), r(beta_1), r(mean_1), r(var_1))
